# Initial kernel scaffold; baseline (speedup 1.0000x reference)
import jax
import jax.numpy as jnp
from jax.experimental import pallas as pl


def kernel(x, edge_index, batch, W1, b1, W2, b2, Wl, bl):
    raise NotImplementedError("write your pallas kernel here")


# speedup vs baseline: 30.2970x; 30.2970x over previous
import functools

import jax
import jax.numpy as jnp
from jax import lax
from jax.experimental import pallas as pl
from jax.experimental.pallas import tpu as pltpu
from jax.experimental.pallas import tpu_sc as plsc

N = 10000
E = 320000
F_IN = 128
H = 64
G = 64

NC = 2
NS = 16
NW = NC * NS
EPW = E // NW
CH = 80
NCH = EPW // CH
RPT = N // NS
DW = 16


@functools.cache
def _sc_kernels():
    mesh = plsc.VectorSubcoreMesh(
        core_axis_name="c", subcore_axis_name="s",
        num_cores=NC, num_subcores=NS)

    no_tc_tiling = pltpu.CompilerParams(use_tc_tiling_on_sc=False)

    @functools.partial(
        pl.kernel,
        out_type=jax.ShapeDtypeStruct((NC, NS, RPT, DW), jnp.float32),
        mesh=mesh,
        compiler_params=no_tc_tiling,
        scratch_types=[
            pltpu.VMEM_SHARED((N, DW), jnp.float32),
            pltpu.VMEM((NCH, CH), jnp.int32),
            pltpu.VMEM((CH, DW), jnp.float32),
        ],
    )
    def deg_kernel(col_hbm, zeros_hbm, ones_hbm, out_hbm, acc, col_v, ones_v):
        c = lax.axis_index("c")
        s = lax.axis_index("s")
        wid = s * NC + c
        pltpu.sync_copy(zeros_hbm.at[s], acc.at[pl.ds(s * RPT, RPT)])
        pltpu.sync_copy(col_hbm.at[wid], col_v)
        pltpu.sync_copy(ones_hbm, ones_v)
        plsc.subcore_barrier()

        def chunk(j, carry):
            pltpu.sync_copy(ones_v, acc.at[col_v.at[j]], add=True)
            return carry

        lax.fori_loop(0, NCH, chunk, 0)
        plsc.subcore_barrier()
        pltpu.sync_copy(acc.at[pl.ds(s * RPT, RPT)], out_hbm.at[c, s])

    @functools.partial(
        pl.kernel,
        out_type=jax.ShapeDtypeStruct((NC, NS, RPT, H), jnp.float32),
        mesh=mesh,
        compiler_params=no_tc_tiling,
        scratch_types=[
            pltpu.VMEM_SHARED((N, H), jnp.float32),
            pltpu.VMEM_SHARED((N, H), jnp.float32),
            pltpu.VMEM((NCH, CH), jnp.int32),
            pltpu.VMEM((NCH, CH), jnp.int32),
            pltpu.VMEM((CH, H), jnp.float32),
            pltpu.SemaphoreType.DMA,
        ],
    )
    def edge_kernel(xws_hbm, row_hbm, col_hbm, zeros_hbm, out_hbm,
                    acc, table, row_v, col_v, rows_v, sem):
        c = lax.axis_index("c")
        s = lax.axis_index("s")
        wid = s * NC + c
        pltpu.sync_copy(zeros_hbm.at[s], acc.at[pl.ds(s * RPT, RPT)])
        pltpu.sync_copy(xws_hbm.at[s], table.at[pl.ds(s * RPT, RPT)])
        pltpu.sync_copy(row_hbm.at[wid], row_v)
        pltpu.sync_copy(col_hbm.at[wid], col_v)
        plsc.subcore_barrier()

        def chunk(j, carry):
            pltpu.async_copy(table.at[row_v.at[j]], rows_v, sem).wait()
            pltpu.sync_copy(rows_v, acc.at[col_v.at[j]], add=True)
            return carry

        lax.fori_loop(0, NCH, chunk, 0)
        plsc.subcore_barrier()
        pltpu.sync_copy(acc.at[pl.ds(s * RPT, RPT)], out_hbm.at[c, s])

    return deg_kernel, edge_kernel


def _dis(deg_ref):
    return lax.rsqrt(deg_ref[0, :, 0:1] + deg_ref[1, :, 0:1] + 1.0)


def _tc_pre_body(x_ref, w_ref, deg_ref, o_ref):
    dis = _dis(deg_ref)
    xw = jnp.dot(x_ref[...], w_ref[...], preferred_element_type=jnp.float32)
    o_ref[...] = xw * dis


def _tc_mid_body(acc_ref, xws_ref, deg_ref, b_ref, w_ref, o_ref):
    dis = _dis(deg_ref)
    pre = dis * (acc_ref[0] + acc_ref[1] + xws_ref[...]) + b_ref[...]
    h = jnp.maximum(pre, 0.0)
    o_ref[...] = jnp.dot(h, w_ref[...], preferred_element_type=jnp.float32) * dis


def _tc_post_body(acc_ref, xws_ref, deg_ref, b_ref, batch_ref, wl_ref, bl_ref,
                  o_ref):
    dis = _dis(deg_ref)
    pre = dis * (acc_ref[0] + acc_ref[1] + xws_ref[...]) + b_ref[...]
    h = jnp.maximum(pre, 0.0)
    gids = lax.broadcasted_iota(jnp.int32, (G, N), 0)
    onehot = (gids == batch_ref[...]).astype(jnp.float32)
    sums = jnp.dot(onehot, h, preferred_element_type=jnp.float32)
    counts = jnp.sum(onehot, axis=1, keepdims=True)
    pooled = sums / jnp.maximum(counts, 1.0)
    logits = jnp.dot(pooled, wl_ref[...], preferred_element_type=jnp.float32)
    o_ref[...] = jax.nn.sigmoid(logits + bl_ref[...])


def _tc_call(body, out_shape, *args):
    return pl.pallas_call(
        body, out_shape=jax.ShapeDtypeStruct(out_shape, jnp.float32))(*args)


@jax.jit
def _impl(x, edge_index, batch, W1, b1, W2, b2, Wl, bl):
    row = edge_index[0].astype(jnp.int32).reshape(NW, NCH, CH)
    col = edge_index[1].astype(jnp.int32).reshape(NW, NCH, CH)
    zeros_nh = jnp.zeros((NS, RPT, H), jnp.float32)
    zeros_nd = jnp.zeros((NS, RPT, DW), jnp.float32)
    ones_cd = jnp.ones((CH, DW), jnp.float32)
    b1r = b1.reshape(1, H)
    b2r = b2.reshape(1, H)
    blr = bl.reshape(1, 1)
    batch_r = batch.astype(jnp.int32).reshape(1, N)

    deg_kernel, edge_kernel = _sc_kernels()
    deg = deg_kernel(col, zeros_nd, ones_cd).reshape(NC, N, DW)
    xws1 = _tc_call(_tc_pre_body, (N, H), x, W1, deg)
    acc1 = edge_kernel(xws1.reshape(NS, RPT, H), row, col,
                       zeros_nh).reshape(NC, N, H)
    xws2 = _tc_call(_tc_mid_body, (N, H), acc1, xws1, deg, b1r, W2)
    acc2 = edge_kernel(xws2.reshape(NS, RPT, H), row, col,
                       zeros_nh).reshape(NC, N, H)
    out = _tc_call(_tc_post_body, (G, 1),
                   acc2, xws2, deg, b2r, batch_r, Wl, blr)
    return out.reshape(G)


def kernel(x, edge_index, batch, W1, b1, W2, b2, Wl, bl):
    return _impl(x, edge_index, batch, W1, b1, W2, b2, Wl, bl)


# speedup vs baseline: 35.9189x; 1.1856x over previous
import functools

import jax
import jax.numpy as jnp
from jax import lax
from jax.experimental import pallas as pl
from jax.experimental.pallas import tpu as pltpu
from jax.experimental.pallas import tpu_sc as plsc

N = 10000
E = 320000
F_IN = 128
H = 64
G = 64

NC = 2
NS = 16
NW = NC * NS
EPW = E // NW
CH = 80
NCH = EPW // CH
RPT = N // NS
DW = 16


@functools.cache
def _sc_kernels():
    mesh = plsc.VectorSubcoreMesh(
        core_axis_name="c", subcore_axis_name="s",
        num_cores=NC, num_subcores=NS)

    no_tc_tiling = pltpu.CompilerParams(use_tc_tiling_on_sc=False)

    @functools.partial(
        pl.kernel,
        out_type=jax.ShapeDtypeStruct((NC, NS, RPT, DW), jnp.float32),
        mesh=mesh,
        compiler_params=no_tc_tiling,
        scratch_types=[
            pltpu.VMEM_SHARED((N, DW), jnp.float32),
            pltpu.VMEM((NCH, CH), jnp.int32),
            pltpu.VMEM((CH, DW), jnp.float32),
        ],
    )
    def deg_kernel(col_hbm, zeros_hbm, ones_hbm, out_hbm, acc, col_v, ones_v):
        c = lax.axis_index("c")
        s = lax.axis_index("s")
        wid = s * NC + c
        pltpu.sync_copy(zeros_hbm.at[s], acc.at[pl.ds(s * RPT, RPT)])
        pltpu.sync_copy(col_hbm.at[wid], col_v)
        pltpu.sync_copy(ones_hbm, ones_v)
        plsc.subcore_barrier()

        def chunk(j, carry):
            pltpu.sync_copy(ones_v, acc.at[col_v.at[j]], add=True)
            return carry

        lax.fori_loop(0, NCH, chunk, 0)
        plsc.subcore_barrier()
        pltpu.sync_copy(acc.at[pl.ds(s * RPT, RPT)], out_hbm.at[c, s])

    @functools.partial(
        pl.kernel,
        out_type=jax.ShapeDtypeStruct((NC, NS, RPT, H), jnp.float32),
        mesh=mesh,
        compiler_params=no_tc_tiling,
        scratch_types=[
            pltpu.VMEM_SHARED((N, H), jnp.float32),
            pltpu.VMEM_SHARED((N, H), jnp.float32),
            pltpu.VMEM((NCH, CH), jnp.int32),
            pltpu.VMEM((NCH, CH), jnp.int32),
            pltpu.VMEM((CH, H), jnp.float32),
            pltpu.VMEM((CH, H), jnp.float32),
            pltpu.SemaphoreType.DMA,
            pltpu.SemaphoreType.DMA,
        ],
    )
    def edge_kernel(xws_hbm, row_hbm, col_hbm, zeros_hbm, out_hbm,
                    acc, table, row_v, col_v, rows_a, rows_b, sem_a, sem_b):
        c = lax.axis_index("c")
        s = lax.axis_index("s")
        wid = s * NC + c
        pltpu.sync_copy(zeros_hbm.at[s], acc.at[pl.ds(s * RPT, RPT)])
        pltpu.sync_copy(xws_hbm.at[s], table.at[pl.ds(s * RPT, RPT)])
        pltpu.sync_copy(row_hbm.at[wid], row_v)
        pltpu.sync_copy(col_hbm.at[wid], col_v)
        plsc.subcore_barrier()

        pltpu.async_copy(table.at[row_v.at[0]], rows_a, sem_a)

        def pair(k, carry):
            j = 2 * k
            pltpu.async_copy(table.at[row_v.at[j + 1]], rows_b, sem_b)
            pltpu.make_async_copy(table.at[row_v.at[j]], rows_a, sem_a).wait()
            pltpu.sync_copy(rows_a, acc.at[col_v.at[j]], add=True)
            pltpu.async_copy(table.at[row_v.at[j + 2]], rows_a, sem_a)
            pltpu.make_async_copy(table.at[row_v.at[j + 1]], rows_b, sem_b).wait()
            pltpu.sync_copy(rows_b, acc.at[col_v.at[j + 1]], add=True)
            return carry

        lax.fori_loop(0, (NCH - 1) // 2, pair, 0)
        pltpu.make_async_copy(table.at[row_v.at[NCH - 1]], rows_a, sem_a).wait()
        pltpu.sync_copy(rows_a, acc.at[col_v.at[NCH - 1]], add=True)
        plsc.subcore_barrier()
        pltpu.sync_copy(acc.at[pl.ds(s * RPT, RPT)], out_hbm.at[c, s])

    return deg_kernel, edge_kernel


def _dis(deg_ref):
    return lax.rsqrt(deg_ref[0, :, 0:1] + deg_ref[1, :, 0:1] + 1.0)


def _tc_pre_body(x_ref, w_ref, deg_ref, o_ref):
    dis = _dis(deg_ref)
    xw = jnp.dot(x_ref[...], w_ref[...], preferred_element_type=jnp.float32)
    o_ref[...] = xw * dis


def _tc_mid_body(acc_ref, xws_ref, deg_ref, b_ref, w_ref, o_ref):
    dis = _dis(deg_ref)
    pre = dis * (acc_ref[0] + acc_ref[1] + xws_ref[...]) + b_ref[...]
    h = jnp.maximum(pre, 0.0)
    o_ref[...] = jnp.dot(h, w_ref[...], preferred_element_type=jnp.float32) * dis


def _tc_post_body(acc_ref, xws_ref, deg_ref, b_ref, batch_ref, wl_ref, bl_ref,
                  o_ref):
    dis = _dis(deg_ref)
    pre = dis * (acc_ref[0] + acc_ref[1] + xws_ref[...]) + b_ref[...]
    h = jnp.maximum(pre, 0.0)
    gids = lax.broadcasted_iota(jnp.int32, (G, N), 0)
    onehot = (gids == batch_ref[...]).astype(jnp.float32)
    sums = jnp.dot(onehot, h, preferred_element_type=jnp.float32)
    counts = jnp.sum(onehot, axis=1, keepdims=True)
    pooled = sums / jnp.maximum(counts, 1.0)
    logits = jnp.dot(pooled, wl_ref[...], preferred_element_type=jnp.float32)
    o_ref[...] = jax.nn.sigmoid(logits + bl_ref[...])


def _tc_call(body, out_shape, *args):
    return pl.pallas_call(
        body, out_shape=jax.ShapeDtypeStruct(out_shape, jnp.float32))(*args)


@jax.jit
def _impl(x, edge_index, batch, W1, b1, W2, b2, Wl, bl):
    row = edge_index[0].astype(jnp.int32).reshape(NW, NCH, CH)
    col = edge_index[1].astype(jnp.int32).reshape(NW, NCH, CH)
    zeros_nh = jnp.zeros((NS, RPT, H), jnp.float32)
    zeros_nd = jnp.zeros((NS, RPT, DW), jnp.float32)
    ones_cd = jnp.ones((CH, DW), jnp.float32)
    b1r = b1.reshape(1, H)
    b2r = b2.reshape(1, H)
    blr = bl.reshape(1, 1)
    batch_r = batch.astype(jnp.int32).reshape(1, N)

    deg_kernel, edge_kernel = _sc_kernels()
    deg = deg_kernel(col, zeros_nd, ones_cd).reshape(NC, N, DW)
    xws1 = _tc_call(_tc_pre_body, (N, H), x, W1, deg)
    acc1 = edge_kernel(xws1.reshape(NS, RPT, H), row, col,
                       zeros_nh).reshape(NC, N, H)
    xws2 = _tc_call(_tc_mid_body, (N, H), acc1, xws1, deg, b1r, W2)
    acc2 = edge_kernel(xws2.reshape(NS, RPT, H), row, col,
                       zeros_nh).reshape(NC, N, H)
    out = _tc_call(_tc_post_body, (G, 1),
                   acc2, xws2, deg, b2r, batch_r, Wl, blr)
    return out.reshape(G)


def kernel(x, edge_index, batch, W1, b1, W2, b2, Wl, bl):
    return _impl(x, edge_index, batch, W1, b1, W2, b2, Wl, bl)


# speedup vs baseline: 36.2060x; 1.0080x over previous
import functools

import jax
import jax.numpy as jnp
from jax import lax
from jax.experimental import pallas as pl
from jax.experimental.pallas import tpu as pltpu
from jax.experimental.pallas import tpu_sc as plsc

N = 10000
E = 320000
F_IN = 128
H = 64
G = 64

NC = 2
NS = 16
NW = NC * NS
EPW = E // NW
CH = 80
NCH = EPW // CH
RPT = N // NS
DW = 8


@functools.cache
def _sc_kernels():
    mesh = plsc.VectorSubcoreMesh(
        core_axis_name="c", subcore_axis_name="s",
        num_cores=NC, num_subcores=NS)

    no_tc_tiling = pltpu.CompilerParams(use_tc_tiling_on_sc=False)

    @functools.partial(
        pl.kernel,
        out_type=jax.ShapeDtypeStruct((NC, NS, RPT, DW), jnp.float32),
        mesh=mesh,
        compiler_params=no_tc_tiling,
        scratch_types=[
            pltpu.VMEM_SHARED((N, DW), jnp.float32),
            pltpu.VMEM((NCH, CH), jnp.int32),
            pltpu.VMEM((CH, DW), jnp.float32),
        ],
    )
    def deg_kernel(col_hbm, zeros_hbm, ones_hbm, out_hbm, acc, col_v, ones_v):
        c = lax.axis_index("c")
        s = lax.axis_index("s")
        wid = s * NC + c
        pltpu.sync_copy(zeros_hbm.at[s], acc.at[pl.ds(s * RPT, RPT)])
        pltpu.sync_copy(col_hbm.at[wid], col_v)
        pltpu.sync_copy(ones_hbm, ones_v)
        plsc.subcore_barrier()

        def chunk(j, carry):
            pltpu.sync_copy(ones_v, acc.at[col_v.at[j]], add=True)
            return carry

        lax.fori_loop(0, NCH, chunk, 0)
        plsc.subcore_barrier()
        pltpu.sync_copy(acc.at[pl.ds(s * RPT, RPT)], out_hbm.at[c, s])

    @functools.partial(
        pl.kernel,
        out_type=jax.ShapeDtypeStruct((NC, NS, RPT, H), jnp.float32),
        mesh=mesh,
        compiler_params=no_tc_tiling,
        scratch_types=[
            pltpu.VMEM_SHARED((N, H), jnp.float32),
            pltpu.VMEM_SHARED((N, H), jnp.float32),
            pltpu.VMEM((NCH, CH), jnp.int32),
            pltpu.VMEM((NCH, CH), jnp.int32),
            pltpu.VMEM((CH, H), jnp.float32),
            pltpu.VMEM((CH, H), jnp.float32),
            pltpu.SemaphoreType.DMA,
            pltpu.SemaphoreType.DMA,
        ],
    )
    def edge_kernel(xws_hbm, row_hbm, col_hbm, zeros_hbm, out_hbm,
                    acc, table, row_v, col_v, rows_a, rows_b, sem_a, sem_b):
        c = lax.axis_index("c")
        s = lax.axis_index("s")
        wid = s * NC + c
        pltpu.sync_copy(zeros_hbm.at[s], acc.at[pl.ds(s * RPT, RPT)])
        pltpu.sync_copy(xws_hbm.at[s], table.at[pl.ds(s * RPT, RPT)])
        pltpu.sync_copy(row_hbm.at[wid], row_v)
        pltpu.sync_copy(col_hbm.at[wid], col_v)
        plsc.subcore_barrier()

        pltpu.async_copy(table.at[row_v.at[0]], rows_a, sem_a)

        def pair(k, carry):
            j = 2 * k
            pltpu.async_copy(table.at[row_v.at[j + 1]], rows_b, sem_b)
            pltpu.make_async_copy(table.at[row_v.at[j]], rows_a, sem_a).wait()
            pltpu.sync_copy(rows_a, acc.at[col_v.at[j]], add=True)
            pltpu.async_copy(table.at[row_v.at[j + 2]], rows_a, sem_a)
            pltpu.make_async_copy(table.at[row_v.at[j + 1]], rows_b, sem_b).wait()
            pltpu.sync_copy(rows_b, acc.at[col_v.at[j + 1]], add=True)
            return carry

        lax.fori_loop(0, (NCH - 1) // 2, pair, 0)
        pltpu.make_async_copy(table.at[row_v.at[NCH - 1]], rows_a, sem_a).wait()
        pltpu.sync_copy(rows_a, acc.at[col_v.at[NCH - 1]], add=True)
        plsc.subcore_barrier()
        pltpu.sync_copy(acc.at[pl.ds(s * RPT, RPT)], out_hbm.at[c, s])

    return deg_kernel, edge_kernel


def _dis(deg_ref):
    return lax.rsqrt(deg_ref[0, :, 0:1] + deg_ref[1, :, 0:1] + 1.0)


def _tc_pre_body(x_ref, w_ref, deg_ref, o_ref):
    dis = _dis(deg_ref)
    xw = jnp.dot(x_ref[...], w_ref[...], preferred_element_type=jnp.float32)
    o_ref[...] = xw * dis


def _tc_mid_body(acc_ref, xws_ref, deg_ref, b_ref, w_ref, o_ref):
    dis = _dis(deg_ref)
    pre = dis * (acc_ref[0] + acc_ref[1] + xws_ref[...]) + b_ref[...]
    h = jnp.maximum(pre, 0.0)
    o_ref[...] = jnp.dot(h, w_ref[...], preferred_element_type=jnp.float32) * dis


def _tc_post_body(acc_ref, xws_ref, deg_ref, b_ref, batch_ref, wl_ref, bl_ref,
                  o_ref):
    dis = _dis(deg_ref)
    pre = dis * (acc_ref[0] + acc_ref[1] + xws_ref[...]) + b_ref[...]
    h = jnp.maximum(pre, 0.0)
    gids = lax.broadcasted_iota(jnp.int32, (G, N), 0)
    onehot = (gids == batch_ref[...]).astype(jnp.float32)
    sums = jnp.dot(onehot, h, preferred_element_type=jnp.float32)
    counts = jnp.sum(onehot, axis=1, keepdims=True)
    pooled = sums / jnp.maximum(counts, 1.0)
    logits = jnp.dot(pooled, wl_ref[...], preferred_element_type=jnp.float32)
    o_ref[...] = jax.nn.sigmoid(logits + bl_ref[...])


def _tc_call(body, out_shape, *args):
    return pl.pallas_call(
        body, out_shape=jax.ShapeDtypeStruct(out_shape, jnp.float32))(*args)


@jax.jit
def _impl(x, edge_index, batch, W1, b1, W2, b2, Wl, bl):
    row = edge_index[0].astype(jnp.int32).reshape(NW, NCH, CH)
    col = edge_index[1].astype(jnp.int32).reshape(NW, NCH, CH)
    zeros_nh = jnp.zeros((NS, RPT, H), jnp.float32)
    zeros_nd = jnp.zeros((NS, RPT, DW), jnp.float32)
    ones_cd = jnp.ones((CH, DW), jnp.float32)
    b1r = b1.reshape(1, H)
    b2r = b2.reshape(1, H)
    blr = bl.reshape(1, 1)
    batch_r = batch.astype(jnp.int32).reshape(1, N)

    deg_kernel, edge_kernel = _sc_kernels()
    deg = deg_kernel(col, zeros_nd, ones_cd).reshape(NC, N, DW)
    xws1 = _tc_call(_tc_pre_body, (N, H), x, W1, deg)
    acc1 = edge_kernel(xws1.reshape(NS, RPT, H), row, col,
                       zeros_nh).reshape(NC, N, H)
    xws2 = _tc_call(_tc_mid_body, (N, H), acc1, xws1, deg, b1r, W2)
    acc2 = edge_kernel(xws2.reshape(NS, RPT, H), row, col,
                       zeros_nh).reshape(NC, N, H)
    out = _tc_call(_tc_post_body, (G, 1),
                   acc2, xws2, deg, b2r, batch_r, Wl, blr)
    return out.reshape(G)


def kernel(x, edge_index, batch, W1, b1, W2, b2, Wl, bl):
    return _impl(x, edge_index, batch, W1, b1, W2, b2, Wl, bl)


# speedup vs baseline: 36.2569x; 1.0014x over previous
import functools

import jax
import jax.numpy as jnp
from jax import lax
from jax.experimental import pallas as pl
from jax.experimental.pallas import tpu as pltpu
from jax.experimental.pallas import tpu_sc as plsc

N = 10000
E = 320000
F_IN = 128
H = 64
G = 64

NC = 2
NS = 16
NW = NC * NS
EPW = E // NW
CH = 80
NCH = EPW // CH
RPT = N // NS
DW = 8


@functools.cache
def _sc_kernels():
    mesh = plsc.VectorSubcoreMesh(
        core_axis_name="c", subcore_axis_name="s",
        num_cores=NC, num_subcores=NS)

    no_tc_tiling = pltpu.CompilerParams(use_tc_tiling_on_sc=False)

    @functools.partial(
        pl.kernel,
        out_type=jax.ShapeDtypeStruct((NC, N, DW), jnp.float32),
        mesh=mesh,
        compiler_params=no_tc_tiling,
        scratch_types=[
            pltpu.VMEM_SHARED((N, DW), jnp.float32),
            pltpu.VMEM((NCH, CH), jnp.int32),
            pltpu.VMEM((CH, DW), jnp.float32),
        ],
    )
    def deg_kernel(col_hbm, zeros_hbm, ones_hbm, out_hbm, acc, col_v, ones_v):
        c = lax.axis_index("c")
        s = lax.axis_index("s")
        wid = s * NC + c
        pltpu.sync_copy(zeros_hbm.at[pl.ds(s * RPT, RPT)],
                        acc.at[pl.ds(s * RPT, RPT)])
        pltpu.sync_copy(col_hbm.at[wid], col_v)
        pltpu.sync_copy(ones_hbm, ones_v)
        plsc.subcore_barrier()

        def chunk(j, carry):
            pltpu.sync_copy(ones_v, acc.at[col_v.at[j]], add=True)
            return carry

        lax.fori_loop(0, NCH, chunk, 0)
        plsc.subcore_barrier()
        pltpu.sync_copy(acc.at[pl.ds(s * RPT, RPT)],
                        out_hbm.at[c, pl.ds(s * RPT, RPT)])

    @functools.partial(
        pl.kernel,
        out_type=jax.ShapeDtypeStruct((NC, N, H), jnp.float32),
        mesh=mesh,
        compiler_params=no_tc_tiling,
        scratch_types=[
            pltpu.VMEM_SHARED((N, H), jnp.float32),
            pltpu.VMEM_SHARED((N, H), jnp.float32),
            pltpu.VMEM((NCH, CH), jnp.int32),
            pltpu.VMEM((NCH, CH), jnp.int32),
            pltpu.VMEM((CH, H), jnp.float32),
            pltpu.VMEM((CH, H), jnp.float32),
            pltpu.SemaphoreType.DMA,
            pltpu.SemaphoreType.DMA,
        ],
    )
    def edge_kernel(xws_hbm, row_hbm, col_hbm, zeros_hbm, out_hbm,
                    acc, table, row_v, col_v, rows_a, rows_b, sem_a, sem_b):
        c = lax.axis_index("c")
        s = lax.axis_index("s")
        wid = s * NC + c
        pltpu.sync_copy(zeros_hbm.at[pl.ds(s * RPT, RPT)],
                        acc.at[pl.ds(s * RPT, RPT)])
        pltpu.sync_copy(xws_hbm.at[pl.ds(s * RPT, RPT)],
                        table.at[pl.ds(s * RPT, RPT)])
        pltpu.sync_copy(row_hbm.at[wid], row_v)
        pltpu.sync_copy(col_hbm.at[wid], col_v)
        plsc.subcore_barrier()

        pltpu.async_copy(table.at[row_v.at[0]], rows_a, sem_a)

        def pair(k, carry):
            j = 2 * k
            pltpu.async_copy(table.at[row_v.at[j + 1]], rows_b, sem_b)
            pltpu.make_async_copy(table.at[row_v.at[j]], rows_a, sem_a).wait()
            pltpu.sync_copy(rows_a, acc.at[col_v.at[j]], add=True)
            pltpu.async_copy(table.at[row_v.at[j + 2]], rows_a, sem_a)
            pltpu.make_async_copy(table.at[row_v.at[j + 1]], rows_b, sem_b).wait()
            pltpu.sync_copy(rows_b, acc.at[col_v.at[j + 1]], add=True)
            return carry

        lax.fori_loop(0, (NCH - 1) // 2, pair, 0)
        pltpu.make_async_copy(table.at[row_v.at[NCH - 1]], rows_a, sem_a).wait()
        pltpu.sync_copy(rows_a, acc.at[col_v.at[NCH - 1]], add=True)
        plsc.subcore_barrier()
        pltpu.sync_copy(acc.at[pl.ds(s * RPT, RPT)],
                        out_hbm.at[c, pl.ds(s * RPT, RPT)])

    return deg_kernel, edge_kernel


def _dis(deg_ref):
    return lax.rsqrt(deg_ref[0, :, 0:1] + deg_ref[1, :, 0:1] + 1.0)


def _tc_pre_body(x_ref, w_ref, deg_ref, o_ref):
    dis = _dis(deg_ref)
    xw = jnp.dot(x_ref[...], w_ref[...], preferred_element_type=jnp.float32)
    o_ref[...] = xw * dis


def _tc_mid_body(acc_ref, xws_ref, deg_ref, b_ref, w_ref, o_ref):
    dis = _dis(deg_ref)
    pre = dis * (acc_ref[0] + acc_ref[1] + xws_ref[...]) + b_ref[...]
    h = jnp.maximum(pre, 0.0)
    o_ref[...] = jnp.dot(h, w_ref[...], preferred_element_type=jnp.float32) * dis


def _tc_post_body(acc_ref, xws_ref, deg_ref, b_ref, batch_ref, wl_ref, bl_ref,
                  o_ref):
    dis = _dis(deg_ref)
    pre = dis * (acc_ref[0] + acc_ref[1] + xws_ref[...]) + b_ref[...]
    h = jnp.maximum(pre, 0.0)
    gids = lax.broadcasted_iota(jnp.int32, (G, N), 0)
    onehot = (gids == batch_ref[...]).astype(jnp.float32)
    sums = jnp.dot(onehot, h, preferred_element_type=jnp.float32)
    counts = jnp.sum(onehot, axis=1, keepdims=True)
    pooled = sums / jnp.maximum(counts, 1.0)
    logits = jnp.dot(pooled, wl_ref[...], preferred_element_type=jnp.float32)
    o_ref[...] = jax.nn.sigmoid(logits + bl_ref[...])


def _tc_call(body, out_shape, *args):
    return pl.pallas_call(
        body, out_shape=jax.ShapeDtypeStruct(out_shape, jnp.float32))(*args)


@jax.jit
def _impl(x, edge_index, batch, W1, b1, W2, b2, Wl, bl):
    row = edge_index[0].astype(jnp.int32).reshape(NW, NCH, CH)
    col = edge_index[1].astype(jnp.int32).reshape(NW, NCH, CH)
    zeros_nh = jnp.zeros((N, H), jnp.float32)
    zeros_nd = jnp.zeros((N, DW), jnp.float32)
    ones_cd = jnp.ones((CH, DW), jnp.float32)
    b1r = b1.reshape(1, H)
    b2r = b2.reshape(1, H)
    blr = bl.reshape(1, 1)
    batch_r = batch.astype(jnp.int32).reshape(1, N)

    deg_kernel, edge_kernel = _sc_kernels()
    deg = deg_kernel(col, zeros_nd, ones_cd)
    xws1 = _tc_call(_tc_pre_body, (N, H), x, W1, deg)
    acc1 = edge_kernel(xws1, row, col, zeros_nh)
    xws2 = _tc_call(_tc_mid_body, (N, H), acc1, xws1, deg, b1r, W2)
    acc2 = edge_kernel(xws2, row, col, zeros_nh)
    out = _tc_call(_tc_post_body, (G, 1),
                   acc2, xws2, deg, b2r, batch_r, Wl, blr)
    return out.reshape(G)


def kernel(x, edge_index, batch, W1, b1, W2, b2, Wl, bl):
    return _impl(x, edge_index, batch, W1, b1, W2, b2, Wl, bl)
